# baseline (speedup 1.0000x reference)
import jax
import jax.numpy as jnp
import numpy as np
from jax.experimental import pallas as pl
from jax.experimental.pallas import tpu as pltpu

B, N, D = 4, 4096, 768
H = 4
HD = D // H
KP = N // 16
KN = N // 32
KQ = KP + KN
RSQD = 1.0 / np.sqrt(D)
LANES = 128
NC = N // LANES
CH = 1024
NCH = N // CH
RB = 512
NRB = N // RB

_PREC = jax.lax.Precision.HIGHEST


def _dot(a, b):
    return jax.lax.dot_general(a, b, (((1,), (0,)), ((), ())),
                               precision=_PREC,
                               preferred_element_type=jnp.float32)


def _dot_nt(a, b):
    return jax.lax.dot_general(a, b, (((1,), (1,)), ((), ())),
                               precision=_PREC,
                               preferred_element_type=jnp.float32)


def _softmax(s):
    m = jnp.max(s, axis=-1, keepdims=True)
    e = jnp.exp(s - m)
    return e / jnp.sum(e, axis=-1, keepdims=True)


def _ln(x, g, b):
    mu = jnp.mean(x, axis=-1, keepdims=True)
    xc = x - mu
    var = jnp.mean(xc * xc, axis=-1, keepdims=True)
    return xc * jax.lax.rsqrt(var + 1e-5) * g + b


def _sigmoid(z):
    return 1.0 / (1.0 + jnp.exp(-z))


def _lane_cumsum(row):
    chunks = [row[:, c * LANES:(c + 1) * LANES] for c in range(NC)]
    m = jnp.concatenate(chunks, axis=0)
    li = jax.lax.broadcasted_iota(jnp.int32, (LANES, LANES), 0)
    lj = jax.lax.broadcasted_iota(jnp.int32, (LANES, LANES), 1)
    u = (li <= lj).astype(jnp.float32)
    cs = _dot(m, u)
    s = jnp.sum(m, axis=-1, keepdims=True)
    ti = jax.lax.broadcasted_iota(jnp.int32, (NC, NC), 0)
    tj = jax.lax.broadcasted_iota(jnp.int32, (NC, NC), 1)
    t = (tj < ti).astype(jnp.float32)
    offs = _dot(t, s)
    full = cs + offs
    return jnp.concatenate([full[c:c + 1, :] for c in range(NC)], axis=1)


def _select(sim, k):
    bits = jax.lax.bitcast_convert_type(sim, jnp.int32)

    def body(_, lohi):
        lo, hi = lohi
        mid = (lo + hi) // 2
        c = jnp.sum((bits >= mid).astype(jnp.int32))
        big = c >= k
        return jnp.where(big, mid, lo), jnp.where(big, hi, mid)

    lo, _ = jax.lax.fori_loop(0, 31, body,
                              (jnp.int32(0), jnp.int32(0x3F800001)))
    gt = bits > lo
    n_gt = jnp.sum(gt.astype(jnp.int32))
    tie = bits == lo
    tie_rank = _lane_cumsum(tie.astype(jnp.float32))
    need = (k - n_gt).astype(jnp.float32)
    sel = jnp.logical_or(gt, jnp.logical_and(tie, tie_rank <= need))
    self32 = sel.astype(jnp.float32)
    rank = _lane_cumsum(self32) - self32
    return self32, rank


def _k0_body(x_ref, ln_ref, fbn2_ref, fbn3_ref):
    xc = x_ref[0]
    ln = ln_ref[...]
    fbn2_ref[0] = _ln(xc, ln[2:3, :], ln[3:4, :])
    fbn3_ref[0] = _ln(xc, ln[4:5, :], ln[5:6, :])


def _k1_body(x_ref, mask_ref, sw_ref,
             psim_ref, nsim_ref, selp_ref, seln_ref, rkp_ref, rkn_ref):
    m = mask_ref[0]
    sw = sw_ref[...]

    pm = (m == 1).astype(jnp.float32)
    nm = (m == -1).astype(jnp.float32)
    pq = jnp.zeros((1, D), jnp.float32)
    nq = jnp.zeros((1, D), jnp.float32)
    for c in range(NCH):
        sl = pl.ds(c * CH, CH)
        xc = x_ref[0, sl, :]
        pq = pq + _dot(pm[:, c * CH:(c + 1) * CH], xc)
        nq = nq + _dot(nm[:, c * CH:(c + 1) * CH], xc)
    pq = pq / (jnp.sum(pm) + 1e-6)
    nq = nq / (jnp.sum(nm) + 1e-6)
    vp = pq * sw[0:1, :]
    vn = nq * sw[1:2, :]
    zp = []
    zn = []
    for c in range(NCH):
        xc = x_ref[0, pl.ds(c * CH, CH), :]
        zp.append(_dot_nt(vp, xc))
        zn.append(_dot_nt(vn, xc))
    bp = jax.lax.broadcast_in_dim(sw[2:3, 0:1], (1, N), (0, 1))
    bn = jax.lax.broadcast_in_dim(sw[3:4, 0:1], (1, N), (0, 1))
    psim = _sigmoid(jnp.concatenate(zp, axis=1) + bp)
    nsim = _sigmoid(jnp.concatenate(zn, axis=1) + bn)
    psim_ref[0] = psim
    nsim_ref[0] = nsim
    selp, rkp = _select(psim, KP)
    seln, rkn = _select(nsim, KN)
    selp_ref[0] = selp
    seln_ref[0] = seln
    rkp_ref[0] = rkp
    rkn_ref[0] = rkn


def _k2_body(x_ref, pe_ref, selp_ref, rkp_ref, psim_ref,
             seln_ref, rkn_ref, nsim_ref,
             adj_ref, w1p_ref, w2p_ref, w1n_ref, w2n_ref, q_ref):
    adj_w = adj_ref[...]

    def branch(sel_ref, rk_ref, sim_ref, k, w1_ref, w2_ref):
        nodes = jnp.zeros((k, D), jnp.float32)
        emb = jnp.zeros((k, D), jnp.float32)
        jrow = jax.lax.broadcasted_iota(jnp.int32, (k, CH), 0)
        for c in range(NCH):
            sl = pl.ds(c * CH, CH)
            sel = sel_ref[0, 0:1, sl]
            rk = rk_ref[0, 0:1, sl]
            sim = sim_ref[0, 0:1, sl]
            on = jnp.logical_and(sel > 0.5,
                                 rk.astype(jnp.int32) == jrow).astype(jnp.float32)
            wfl = jnp.logical_and(sel > 0.5, sim > 0.6).astype(jnp.float32)
            nodes = nodes + _dot(on * wfl, x_ref[0, sl, :])
            emb = emb + _dot(on, pe_ref[0, sl, :])
        t = _dot(nodes, adj_w)
        a = _softmax(_dot_nt(t, nodes) * RSQD)
        h = jnp.maximum(_dot(a, _dot(nodes, w1_ref[...])), 0.0)
        return _dot(a, _dot(h, w2_ref[...])) + emb

    q_ref[0, 0:KP, :] = branch(selp_ref, rkp_ref, psim_ref, KP,
                               w1p_ref, w2p_ref)
    q_ref[0, KP:KQ, :] = branch(seln_ref, rkn_ref, nsim_ref, KN,
                                w1n_ref, w2n_ref)


def _k3_body(q_ref, fbn2_ref, ln_ref, wq_ref, wk_ref, wv_ref, wo_ref,
             bo_ref, g_ref,
             out_ref, qh_scr, acc_scr, ml_scr):
    c = pl.program_id(1)
    ln = ln_ref[...]
    scale = HD ** -0.5

    @pl.when(c == 0)
    def _init():
        qn = _ln(q_ref[0], ln[0:1, :], ln[1:2, :])
        for h in range(H):
            qh_scr[h] = _dot(qn, wq_ref[h])
            acc_scr[h] = jnp.zeros((KQ, HD), jnp.float32)
        ml_scr[0] = jnp.full((H, KQ, LANES), -1e30, jnp.float32)
        ml_scr[1] = jnp.zeros((H, KQ, LANES), jnp.float32)

    fc = fbn2_ref[0]
    for h in range(H):
        kh = _dot(fc, wk_ref[h])
        vh = _dot(fc, wv_ref[h])
        s = _dot_nt(qh_scr[h], kh) * scale
        m_old = ml_scr[0, h, :, 0:1]
        l_old = ml_scr[1, h, :, 0:1]
        m_new = jnp.maximum(m_old, jnp.max(s, axis=-1, keepdims=True))
        alpha = jnp.exp(m_old - m_new)
        p = jnp.exp(s - m_new)
        l_new = l_old * alpha + jnp.sum(p, axis=-1, keepdims=True)
        acc_scr[h] = acc_scr[h] * alpha + _dot(p, vh)
        ml_scr[0, h] = jnp.broadcast_to(m_new, (KQ, LANES))
        ml_scr[1, h] = jnp.broadcast_to(l_new, (KQ, LANES))

    @pl.when(c == NCH - 1)
    def _fin():
        attn = jnp.zeros((KQ, D), jnp.float32)
        for h in range(H):
            attn = attn + _dot(acc_scr[h] / ml_scr[1, h, :, 0:1], wo_ref[h])
        q2 = q_ref[0] + g_ref[...] * (attn + bo_ref[...])
        out_ref[0] = q2


def _k3b_body(q_ref, ln_ref, kv_wk_ref, kv_wv_ref, kq_ref, vq_ref):
    ln = ln_ref[...]
    qn4 = _ln(q_ref[0], ln[6:7, :], ln[7:8, :])
    for h in range(H):
        kq_ref[0, h] = _dot(qn4, kv_wk_ref[h])
        vq_ref[0, h] = _dot(qn4, kv_wv_ref[h])


def _k4_body(x_ref, fbn3_ref, kq_ref, vq_ref, ln_ref, wq_ref, wo_ref,
             bo_ref, g_ref, mlpw_ref, mlpb_ref, out_ref):
    fb = x_ref[0]
    fbn = fbn3_ref[0]
    ln = ln_ref[...]
    scale = HD ** -0.5
    attn = jnp.zeros((RB, D), jnp.float32)
    for h in range(H):
        qh = _dot(fbn, wq_ref[h])
        p = _softmax(_dot_nt(qh, kq_ref[0, h]) * scale)
        attn = attn + _dot(_dot(p, vq_ref[0, h]), wo_ref[h])
    fb2 = fb + g_ref[...] * (attn + bo_ref[...])
    mlp = _dot(_ln(fb2, ln[8:9, :], ln[9:10, :]), mlpw_ref[...]) + mlpb_ref[...]
    out_ref[0, 0] = fb2 + mlp


def _cp():
    return pltpu.CompilerParams(vmem_limit_bytes=62 * 1024 * 1024)


def _row_spec():
    return pl.BlockSpec((1, 1, N), lambda b: (b, 0, 0))


def _const(shape, ngrid=1):
    nd = len(shape)
    if ngrid == 1:
        return pl.BlockSpec(shape, lambda b, _n=nd: (0,) * _n)
    return pl.BlockSpec(shape, lambda b, r, _n=nd: (0,) * _n)


def _heads(w):
    return w.reshape(D, H, HD).transpose(1, 0, 2)


def kernel(x, mask, pos_emb, params):
    p = params
    lnstack = jnp.stack([p['ln1_g'], p['ln1_b'], p['ln2_g'], p['ln2_b'],
                         p['ln3_g'], p['ln3_b'], p['ln4_g'], p['ln4_b'],
                         p['ln5_g'], p['ln5_b']])
    sw = jnp.concatenate([
        p['simp_w'].T, p['simn_w'].T,
        jnp.broadcast_to(p['simp_b'].reshape(1, 1), (1, D)),
        jnp.broadcast_to(p['simn_b'].reshape(1, 1), (1, D)),
    ], axis=0)

    nd_f = jax.ShapeDtypeStruct((B, N, D), jnp.float32)
    fbn2, fbn3 = pl.pallas_call(
        _k0_body,
        grid=(B, NRB),
        in_specs=[pl.BlockSpec((1, RB, D), lambda b, r: (b, r, 0)),
                  _const((10, D), 2)],
        out_specs=[pl.BlockSpec((1, RB, D), lambda b, r: (b, r, 0))] * 2,
        out_shape=[nd_f] * 2,
        compiler_params=_cp(),
    )(x, lnstack)

    rowf = jax.ShapeDtypeStruct((B, 1, N), jnp.float32)
    psim, nsim, selp, seln, rkp, rkn = pl.pallas_call(
        _k1_body,
        grid=(B,),
        in_specs=[pl.BlockSpec((1, N, D), lambda b: (b, 0, 0)),
                  _row_spec(), _const((4, D))],
        out_specs=[_row_spec()] * 6,
        out_shape=[rowf] * 6,
        compiler_params=_cp(),
    )(x, mask.reshape(B, 1, N), sw)

    query = pl.pallas_call(
        _k2_body,
        grid=(B,),
        in_specs=[pl.BlockSpec((1, N, D), lambda b: (b, 0, 0)),
                  _const((1, N, D)),
                  _row_spec(), _row_spec(), _row_spec(),
                  _row_spec(), _row_spec(), _row_spec(),
                  _const((D, D)), _const((D, D)), _const((D, D)),
                  _const((D, D)), _const((D, D))],
        out_specs=pl.BlockSpec((1, KQ, D), lambda b: (b, 0, 0)),
        out_shape=jax.ShapeDtypeStruct((B, KQ, D), jnp.float32),
        compiler_params=_cp(),
    )(x, pos_emb, selp, rkp, psim, seln, rkn, nsim,
      p['adj_w'], p['gnnp_w1'], p['gnnp_w2'], p['gnnn_w1'], p['gnnn_w2'])

    query2 = pl.pallas_call(
        _k3_body,
        grid=(B, NCH),
        in_specs=[pl.BlockSpec((1, KQ, D), lambda b, c: (b, 0, 0)),
                  pl.BlockSpec((1, CH, D), lambda b, c: (b, c, 0)),
                  _const((10, D), 2),
                  _const((H, D, HD), 2), _const((H, D, HD), 2),
                  _const((H, D, HD), 2),
                  _const((H, HD, D), 2), _const((1, D), 2), _const((1, D), 2)],
        out_specs=pl.BlockSpec((1, KQ, D), lambda b, c: (b, 0, 0)),
        out_shape=jax.ShapeDtypeStruct((B, KQ, D), jnp.float32),
        scratch_shapes=[pltpu.VMEM((H, KQ, HD), jnp.float32),
                        pltpu.VMEM((H, KQ, HD), jnp.float32),
                        pltpu.VMEM((2, H, KQ, LANES), jnp.float32)],
        compiler_params=_cp(),
    )(query, fbn2, lnstack,
      _heads(p['i2q_wq']), _heads(p['i2q_wk']), _heads(p['i2q_wv']),
      p['i2q_wo'].reshape(H, HD, D), p['i2q_bo'].reshape(1, D),
      p['g_i2t'].reshape(1, D))

    kq, vq = pl.pallas_call(
        _k3b_body,
        grid=(B,),
        in_specs=[pl.BlockSpec((1, KQ, D), lambda b: (b, 0, 0)),
                  _const((10, D)),
                  _const((H, D, HD)), _const((H, D, HD))],
        out_specs=[pl.BlockSpec((1, H, KQ, HD), lambda b: (b, 0, 0, 0))] * 2,
        out_shape=[jax.ShapeDtypeStruct((B, H, KQ, HD), jnp.float32)] * 2,
        compiler_params=_cp(),
    )(query2, lnstack, _heads(p['q2i_wk']), _heads(p['q2i_wv']))

    xo = pl.pallas_call(
        _k4_body,
        grid=(B, NRB),
        in_specs=[pl.BlockSpec((1, RB, D), lambda b, r: (b, r, 0)),
                  pl.BlockSpec((1, RB, D), lambda b, r: (b, r, 0)),
                  pl.BlockSpec((1, H, KQ, HD), lambda b, r: (b, 0, 0, 0)),
                  pl.BlockSpec((1, H, KQ, HD), lambda b, r: (b, 0, 0, 0)),
                  _const((10, D), 2),
                  _const((H, D, HD), 2), _const((H, HD, D), 2),
                  _const((1, D), 2), _const((1, D), 2),
                  _const((D, D), 2), _const((1, D), 2)],
        out_specs=pl.BlockSpec((1, 1, RB, D), lambda b, r: (b, 0, r, 0)),
        out_shape=jax.ShapeDtypeStruct((B, 1, N, D), jnp.float32),
        compiler_params=_cp(),
    )(x, fbn3, kq, vq, lnstack,
      _heads(p['q2i_wq']), p['q2i_wo'].reshape(H, HD, D),
      p['q2i_bo'].reshape(1, D), p['g_t2i'].reshape(1, D),
      p['mlp_w'], p['mlp_b'].reshape(1, D))

    return xo, psim.reshape(B, N), nsim.reshape(B, N)


# speedup vs baseline: 3.4093x; 3.4093x over previous
import jax
import jax.numpy as jnp
import numpy as np
from jax.experimental import pallas as pl
from jax.experimental.pallas import tpu as pltpu

B, N, D = 4, 4096, 768
H = 4
HD = D // H
KP = N // 16
KN = N // 32
KQ = KP + KN
RSQD = 1.0 / np.sqrt(D)
LANES = 128
NC = N // LANES
CH = 1024
NCH = N // CH
RB = 512
NRB = N // RB

_PREC = jax.lax.Precision.DEFAULT
_PRECX = jax.lax.Precision.HIGHEST


def _dot(a, b, prec=_PREC):
    return jax.lax.dot_general(a, b, (((1,), (0,)), ((), ())),
                               precision=prec,
                               preferred_element_type=jnp.float32)


def _dot_nt(a, b, prec=_PREC):
    return jax.lax.dot_general(a, b, (((1,), (1,)), ((), ())),
                               precision=prec,
                               preferred_element_type=jnp.float32)


def _softmax(s):
    m = jnp.max(s, axis=-1, keepdims=True)
    e = jnp.exp(s - m)
    return e / jnp.sum(e, axis=-1, keepdims=True)


def _ln(x, g, b):
    mu = jnp.mean(x, axis=-1, keepdims=True)
    xc = x - mu
    var = jnp.mean(xc * xc, axis=-1, keepdims=True)
    return xc * jax.lax.rsqrt(var + 1e-5) * g + b


def _sigmoid(z):
    return 1.0 / (1.0 + jnp.exp(-z))


def _lane_cumsum(row):
    chunks = [row[:, c * LANES:(c + 1) * LANES] for c in range(NC)]
    m = jnp.concatenate(chunks, axis=0)
    li = jax.lax.broadcasted_iota(jnp.int32, (LANES, LANES), 0)
    lj = jax.lax.broadcasted_iota(jnp.int32, (LANES, LANES), 1)
    u = (li <= lj).astype(jnp.float32)
    cs = _dot(m, u, _PRECX)
    s = jnp.sum(m, axis=-1, keepdims=True)
    ti = jax.lax.broadcasted_iota(jnp.int32, (NC, NC), 0)
    tj = jax.lax.broadcasted_iota(jnp.int32, (NC, NC), 1)
    t = (tj < ti).astype(jnp.float32)
    offs = _dot(t, s, _PRECX)
    full = cs + offs
    return jnp.concatenate([full[c:c + 1, :] for c in range(NC)], axis=1)


def _select(sim, k):
    bits = jax.lax.bitcast_convert_type(sim, jnp.int32)

    def body(_, lohi):
        lo, hi = lohi
        mid = (lo + hi) // 2
        c = jnp.sum((bits >= mid).astype(jnp.int32))
        big = c >= k
        return jnp.where(big, mid, lo), jnp.where(big, hi, mid)

    lo, _ = jax.lax.fori_loop(0, 31, body,
                              (jnp.int32(0), jnp.int32(0x3F800001)))
    gt = bits > lo
    n_gt = jnp.sum(gt.astype(jnp.int32))
    tie = bits == lo
    tie_rank = _lane_cumsum(tie.astype(jnp.float32))
    need = (k - n_gt).astype(jnp.float32)
    sel = jnp.logical_or(gt, jnp.logical_and(tie, tie_rank <= need))
    self32 = sel.astype(jnp.float32)
    rank = _lane_cumsum(self32) - self32
    return self32, rank


def _k0_body(x_ref, ln_ref, fbn2_ref, fbn3_ref):
    xc = x_ref[0]
    ln = ln_ref[...]
    fbn2_ref[0] = _ln(xc, ln[2:3, :], ln[3:4, :])
    fbn3_ref[0] = _ln(xc, ln[4:5, :], ln[5:6, :])


def _k1_body(x_ref, mask_ref, sw_ref,
             psim_ref, nsim_ref, selp_ref, seln_ref, rkp_ref, rkn_ref):
    m = mask_ref[0]
    sw = sw_ref[...]

    pm = (m == 1).astype(jnp.float32)
    nm = (m == -1).astype(jnp.float32)
    pq = jnp.zeros((1, D), jnp.float32)
    nq = jnp.zeros((1, D), jnp.float32)
    for c in range(NCH):
        sl = pl.ds(c * CH, CH)
        xc = x_ref[0, sl, :]
        pq = pq + _dot(pm[:, c * CH:(c + 1) * CH], xc)
        nq = nq + _dot(nm[:, c * CH:(c + 1) * CH], xc)
    pq = pq / (jnp.sum(pm) + 1e-6)
    nq = nq / (jnp.sum(nm) + 1e-6)
    vp = pq * sw[0:1, :]
    vn = nq * sw[1:2, :]
    zp = []
    zn = []
    for c in range(NCH):
        xc = x_ref[0, pl.ds(c * CH, CH), :]
        zp.append(_dot_nt(vp, xc))
        zn.append(_dot_nt(vn, xc))
    bp = jax.lax.broadcast_in_dim(sw[2:3, 0:1], (1, N), (0, 1))
    bn = jax.lax.broadcast_in_dim(sw[3:4, 0:1], (1, N), (0, 1))
    psim = _sigmoid(jnp.concatenate(zp, axis=1) + bp)
    nsim = _sigmoid(jnp.concatenate(zn, axis=1) + bn)
    psim_ref[0] = psim
    nsim_ref[0] = nsim
    selp, rkp = _select(psim, KP)
    seln, rkn = _select(nsim, KN)
    selp_ref[0] = selp
    seln_ref[0] = seln
    rkp_ref[0] = rkp
    rkn_ref[0] = rkn


def _k2_body(x_ref, pe_ref, selp_ref, rkp_ref, psim_ref,
             seln_ref, rkn_ref, nsim_ref,
             adj_ref, w1p_ref, w2p_ref, w1n_ref, w2n_ref, q_ref):
    adj_w = adj_ref[...]

    def branch(sel_ref, rk_ref, sim_ref, k, w1_ref, w2_ref):
        nodes = jnp.zeros((k, D), jnp.float32)
        emb = jnp.zeros((k, D), jnp.float32)
        jrow = jax.lax.broadcasted_iota(jnp.int32, (k, CH), 0)
        for c in range(NCH):
            sl = pl.ds(c * CH, CH)
            sel = sel_ref[0, 0:1, sl]
            rk = rk_ref[0, 0:1, sl]
            sim = sim_ref[0, 0:1, sl]
            on = jnp.logical_and(sel > 0.5,
                                 rk.astype(jnp.int32) == jrow).astype(jnp.float32)
            wfl = jnp.logical_and(sel > 0.5, sim > 0.6).astype(jnp.float32)
            nodes = nodes + _dot(on * wfl, x_ref[0, sl, :], _PRECX)
            emb = emb + _dot(on, pe_ref[0, sl, :], _PRECX)
        t = _dot(nodes, adj_w)
        a = _softmax(_dot_nt(t, nodes) * RSQD)
        h = jnp.maximum(_dot(a, _dot(nodes, w1_ref[...])), 0.0)
        return _dot(a, _dot(h, w2_ref[...])) + emb

    q_ref[0, 0:KP, :] = branch(selp_ref, rkp_ref, psim_ref, KP,
                               w1p_ref, w2p_ref)
    q_ref[0, KP:KQ, :] = branch(seln_ref, rkn_ref, nsim_ref, KN,
                                w1n_ref, w2n_ref)


def _k3_body(q_ref, fbn2_ref, ln_ref, wq_ref, wk_ref, wv_ref, wo_ref,
             bo_ref, g_ref,
             out_ref, qh_scr, acc_scr, ml_scr):
    c = pl.program_id(1)
    ln = ln_ref[...]
    scale = HD ** -0.5

    @pl.when(c == 0)
    def _init():
        qn = _ln(q_ref[0], ln[0:1, :], ln[1:2, :])
        for h in range(H):
            qh_scr[h] = _dot(qn, wq_ref[h])
            acc_scr[h] = jnp.zeros((KQ, HD), jnp.float32)
        ml_scr[0] = jnp.full((H, KQ, LANES), -1e30, jnp.float32)
        ml_scr[1] = jnp.zeros((H, KQ, LANES), jnp.float32)

    fc = fbn2_ref[0]
    for h in range(H):
        kh = _dot(fc, wk_ref[h])
        vh = _dot(fc, wv_ref[h])
        s = _dot_nt(qh_scr[h], kh) * scale
        m_old = ml_scr[0, h, :, 0:1]
        l_old = ml_scr[1, h, :, 0:1]
        m_new = jnp.maximum(m_old, jnp.max(s, axis=-1, keepdims=True))
        alpha = jnp.exp(m_old - m_new)
        p = jnp.exp(s - m_new)
        l_new = l_old * alpha + jnp.sum(p, axis=-1, keepdims=True)
        acc_scr[h] = acc_scr[h] * alpha + _dot(p, vh)
        ml_scr[0, h] = jnp.broadcast_to(m_new, (KQ, LANES))
        ml_scr[1, h] = jnp.broadcast_to(l_new, (KQ, LANES))

    @pl.when(c == NCH - 1)
    def _fin():
        attn = jnp.zeros((KQ, D), jnp.float32)
        for h in range(H):
            attn = attn + _dot(acc_scr[h] / ml_scr[1, h, :, 0:1], wo_ref[h])
        q2 = q_ref[0] + g_ref[...] * (attn + bo_ref[...])
        out_ref[0] = q2


def _k3b_body(q_ref, ln_ref, kv_wk_ref, kv_wv_ref, kq_ref, vq_ref):
    ln = ln_ref[...]
    qn4 = _ln(q_ref[0], ln[6:7, :], ln[7:8, :])
    for h in range(H):
        kq_ref[0, h] = _dot(qn4, kv_wk_ref[h])
        vq_ref[0, h] = _dot(qn4, kv_wv_ref[h])


def _k4_body(x_ref, fbn3_ref, kq_ref, vq_ref, ln_ref, wq_ref, wo_ref,
             bo_ref, g_ref, mlpw_ref, mlpb_ref, out_ref):
    fb = x_ref[0]
    fbn = fbn3_ref[0]
    ln = ln_ref[...]
    scale = HD ** -0.5
    attn = jnp.zeros((RB, D), jnp.float32)
    for h in range(H):
        qh = _dot(fbn, wq_ref[h])
        p = _softmax(_dot_nt(qh, kq_ref[0, h]) * scale)
        attn = attn + _dot(_dot(p, vq_ref[0, h]), wo_ref[h])
    fb2 = fb + g_ref[...] * (attn + bo_ref[...])
    mlp = _dot(_ln(fb2, ln[8:9, :], ln[9:10, :]), mlpw_ref[...]) + mlpb_ref[...]
    out_ref[0, 0] = fb2 + mlp


def _cp():
    return pltpu.CompilerParams(vmem_limit_bytes=62 * 1024 * 1024)


def _row_spec():
    return pl.BlockSpec((1, 1, N), lambda b: (b, 0, 0))


def _const(shape, ngrid=1):
    nd = len(shape)
    if ngrid == 1:
        return pl.BlockSpec(shape, lambda b, _n=nd: (0,) * _n)
    return pl.BlockSpec(shape, lambda b, r, _n=nd: (0,) * _n)


def _heads(w):
    return w.reshape(D, H, HD).transpose(1, 0, 2)


def kernel(x, mask, pos_emb, params):
    p = params
    lnstack = jnp.stack([p['ln1_g'], p['ln1_b'], p['ln2_g'], p['ln2_b'],
                         p['ln3_g'], p['ln3_b'], p['ln4_g'], p['ln4_b'],
                         p['ln5_g'], p['ln5_b']])
    sw = jnp.concatenate([
        p['simp_w'].T, p['simn_w'].T,
        jnp.broadcast_to(p['simp_b'].reshape(1, 1), (1, D)),
        jnp.broadcast_to(p['simn_b'].reshape(1, 1), (1, D)),
    ], axis=0)

    nd_f = jax.ShapeDtypeStruct((B, N, D), jnp.float32)
    fbn2, fbn3 = pl.pallas_call(
        _k0_body,
        grid=(B, NRB),
        in_specs=[pl.BlockSpec((1, RB, D), lambda b, r: (b, r, 0)),
                  _const((10, D), 2)],
        out_specs=[pl.BlockSpec((1, RB, D), lambda b, r: (b, r, 0))] * 2,
        out_shape=[nd_f] * 2,
        compiler_params=_cp(),
    )(x, lnstack)

    rowf = jax.ShapeDtypeStruct((B, 1, N), jnp.float32)
    psim, nsim, selp, seln, rkp, rkn = pl.pallas_call(
        _k1_body,
        grid=(B,),
        in_specs=[pl.BlockSpec((1, N, D), lambda b: (b, 0, 0)),
                  _row_spec(), _const((4, D))],
        out_specs=[_row_spec()] * 6,
        out_shape=[rowf] * 6,
        compiler_params=_cp(),
    )(x, mask.reshape(B, 1, N), sw)

    query = pl.pallas_call(
        _k2_body,
        grid=(B,),
        in_specs=[pl.BlockSpec((1, N, D), lambda b: (b, 0, 0)),
                  _const((1, N, D)),
                  _row_spec(), _row_spec(), _row_spec(),
                  _row_spec(), _row_spec(), _row_spec(),
                  _const((D, D)), _const((D, D)), _const((D, D)),
                  _const((D, D)), _const((D, D))],
        out_specs=pl.BlockSpec((1, KQ, D), lambda b: (b, 0, 0)),
        out_shape=jax.ShapeDtypeStruct((B, KQ, D), jnp.float32),
        compiler_params=_cp(),
    )(x, pos_emb, selp, rkp, psim, seln, rkn, nsim,
      p['adj_w'], p['gnnp_w1'], p['gnnp_w2'], p['gnnn_w1'], p['gnnn_w2'])

    query2 = pl.pallas_call(
        _k3_body,
        grid=(B, NCH),
        in_specs=[pl.BlockSpec((1, KQ, D), lambda b, c: (b, 0, 0)),
                  pl.BlockSpec((1, CH, D), lambda b, c: (b, c, 0)),
                  _const((10, D), 2),
                  _const((H, D, HD), 2), _const((H, D, HD), 2),
                  _const((H, D, HD), 2),
                  _const((H, HD, D), 2), _const((1, D), 2), _const((1, D), 2)],
        out_specs=pl.BlockSpec((1, KQ, D), lambda b, c: (b, 0, 0)),
        out_shape=jax.ShapeDtypeStruct((B, KQ, D), jnp.float32),
        scratch_shapes=[pltpu.VMEM((H, KQ, HD), jnp.float32),
                        pltpu.VMEM((H, KQ, HD), jnp.float32),
                        pltpu.VMEM((2, H, KQ, LANES), jnp.float32)],
        compiler_params=_cp(),
    )(query, fbn2, lnstack,
      _heads(p['i2q_wq']), _heads(p['i2q_wk']), _heads(p['i2q_wv']),
      p['i2q_wo'].reshape(H, HD, D), p['i2q_bo'].reshape(1, D),
      p['g_i2t'].reshape(1, D))

    kq, vq = pl.pallas_call(
        _k3b_body,
        grid=(B,),
        in_specs=[pl.BlockSpec((1, KQ, D), lambda b: (b, 0, 0)),
                  _const((10, D)),
                  _const((H, D, HD)), _const((H, D, HD))],
        out_specs=[pl.BlockSpec((1, H, KQ, HD), lambda b: (b, 0, 0, 0))] * 2,
        out_shape=[jax.ShapeDtypeStruct((B, H, KQ, HD), jnp.float32)] * 2,
        compiler_params=_cp(),
    )(query2, lnstack, _heads(p['q2i_wk']), _heads(p['q2i_wv']))

    xo = pl.pallas_call(
        _k4_body,
        grid=(B, NRB),
        in_specs=[pl.BlockSpec((1, RB, D), lambda b, r: (b, r, 0)),
                  pl.BlockSpec((1, RB, D), lambda b, r: (b, r, 0)),
                  pl.BlockSpec((1, H, KQ, HD), lambda b, r: (b, 0, 0, 0)),
                  pl.BlockSpec((1, H, KQ, HD), lambda b, r: (b, 0, 0, 0)),
                  _const((10, D), 2),
                  _const((H, D, HD), 2), _const((H, HD, D), 2),
                  _const((1, D), 2), _const((1, D), 2),
                  _const((D, D), 2), _const((1, D), 2)],
        out_specs=pl.BlockSpec((1, 1, RB, D), lambda b, r: (b, 0, r, 0)),
        out_shape=jax.ShapeDtypeStruct((B, 1, N, D), jnp.float32),
        compiler_params=_cp(),
    )(x, fbn3, kq, vq, lnstack,
      _heads(p['q2i_wq']), p['q2i_wo'].reshape(H, HD, D),
      p['q2i_bo'].reshape(1, D), p['g_t2i'].reshape(1, D),
      p['mlp_w'], p['mlp_b'].reshape(1, D))

    return xo, psim.reshape(B, N), nsim.reshape(B, N)
